# baseline (speedup 1.0000x reference)
import functools

import jax
import jax.numpy as jnp
from jax import lax
from jax.experimental import pallas as pl
from jax.experimental.pallas import tpu as pltpu
from jax.experimental.pallas import tpu_sc as plsc


def _tc_argmax(xn, cn, mt=256):
    m, d = xn.shape
    k_total = cn.shape[0]
    n_mt = m // mt

    def body(x_ref, cn_ref, idx_ref, ms_ref):
        sim = lax.dot_general(x_ref[...], cn_ref[...], (((1,), (1,)), ((), ())),
                              preferred_element_type=jnp.float32)
        tmax = jnp.max(sim, axis=1)
        targ = jnp.argmax(sim, axis=1).astype(jnp.int32)
        idx_ref[...] = targ
        ms_ref[...] = tmax

    return pl.pallas_call(
        body,
        grid=(n_mt,),
        in_specs=[
            pl.BlockSpec((mt, d), lambda mi: (mi, 0)),
            pl.BlockSpec((k_total, d), lambda mi: (0, 0)),
        ],
        out_specs=[
            pl.BlockSpec((mt,), lambda mi: (mi,)),
            pl.BlockSpec((mt,), lambda mi: (mi,)),
        ],
        out_shape=[
            jax.ShapeDtypeStruct((m,), jnp.int32),
            jax.ShapeDtypeStruct((m,), jnp.float32),
        ],
        compiler_params=pltpu.CompilerParams(
            vmem_limit_bytes=100 * 1024 * 1024),
    )(xn, cn)


def _sc_gather_hist(cn, idx):
    k_total, d = cn.shape
    m = idx.shape[0]
    info = plsc.get_sparse_core_info()
    nc, ns = info.num_cores, info.num_subcores
    nw = nc * ns
    rows_w = m // nw
    ch = 128
    n_ch = rows_w // ch
    hr, hc = k_total // 128, 128

    mesh = plsc.VectorSubcoreMesh(core_axis_name="c", subcore_axis_name="s")

    @functools.partial(
        pl.kernel,
        out_type=[
            jax.ShapeDtypeStruct((m, d), jnp.float32),
            jax.ShapeDtypeStruct((nc, hr, hc), jnp.float32),
        ],
        mesh=mesh,
        compiler_params=pltpu.CompilerParams(needs_layout_passes=False),
        scratch_types=[
            pltpu.VMEM((rows_w,), jnp.int32),
            pltpu.VMEM((ch, d), jnp.float32),
            pltpu.VMEM((ch, d), jnp.float32),
            pltpu.VMEM((hr, hc), jnp.float32),
            pltpu.VMEM((hr,), jnp.int32),
            pltpu.VMEM_SHARED((hr, hc), jnp.float32),
            pltpu.SemaphoreType.DMA,
            pltpu.SemaphoreType.DMA,
        ],
    )
    def sc_body(cn_hbm, idx_hbm, zq_hbm, cnt_hbm,
                idx_v, buf0, buf1, hist_v, rowid_v, shared_hist, sem0, sem1):
        ci = lax.axis_index("c")
        si = lax.axis_index("s")
        wid = si * nc + ci
        base = wid * rows_w

        pltpu.sync_copy(idx_hbm.at[pl.ds(base, rows_w)], idx_v)
        bufs = (buf0, buf1)
        sems = (sem0, sem1)
        cps = [pltpu.async_copy(cn_hbm.at[idx_v.at[pl.ds(0, ch)]], buf0, sem0),
               pltpu.async_copy(cn_hbm.at[idx_v.at[pl.ds(ch, ch)]], buf1, sem1)]

        zeros16 = jnp.zeros((16,), jnp.float32)

        def zero_body(t, _):
            r = t // (hc // 16)
            c = (t % (hc // 16)) * 16
            hist_v[r, pl.ds(c, 16)] = zeros16
            return 0

        lax.fori_loop(0, hr * (hc // 16), zero_body, 0)
        for j in range(hr // 16):
            rowid_v[pl.ds(j * 16, 16)] = lax.iota(jnp.int32, 16) + j * 16

        @pl.when(si == 0)
        def _():
            pltpu.sync_copy(hist_v, shared_hist)

        lane = lax.iota(jnp.int32, 16)
        ones16 = jnp.ones((16,), jnp.float32)

        def hist_body(v, _):
            vec = idx_v[pl.ds(v * 16, 16)]
            row = lax.shift_right_logical(vec, 7)
            col = lax.bitwise_and(vec, 127)
            for j in range(16):
                plsc.addupdate_scatter(hist_v, [row, col], ones16,
                                       mask=lane == j)
            return 0

        lax.fori_loop(0, rows_w // 16, hist_body, 0)

        for c in range(n_ch):
            cps[c % 2].wait()
            pltpu.sync_copy(bufs[c % 2], zq_hbm.at[pl.ds(base + c * ch, ch)])
            if c + 2 < n_ch:
                cps[c % 2] = pltpu.async_copy(
                    cn_hbm.at[idx_v.at[pl.ds((c + 2) * ch, ch)]],
                    bufs[c % 2], sems[c % 2])

        plsc.subcore_barrier()
        pltpu.sync_copy(hist_v, shared_hist.at[rowid_v], add=True)
        plsc.subcore_barrier()

        @pl.when(si == 0)
        def _():
            pltpu.sync_copy(shared_hist, cnt_hbm.at[ci])

    return sc_body(cn, idx)


def _tc_scalars(ms2d, counts, m, d):
    def body(ms_ref, cnt_ref, loss_ref, perp_ref):
        mean_s = jnp.sum(ms_ref[...]) * (1.0 / m)
        loss = 1.25 * (2.0 - 2.0 * mean_s) * (1.0 / d)
        loss_ref[...] = jnp.broadcast_to(loss, (1, 1))
        cnt = cnt_ref[0] + cnt_ref[1]
        e = cnt * (1.0 / m)
        ent = -jnp.sum(e * jnp.log(e + 1e-10))
        perp_ref[...] = jnp.broadcast_to(jnp.exp(ent), (1, 1))

    return pl.pallas_call(
        body,
        out_shape=[
            jax.ShapeDtypeStruct((1, 1), jnp.float32),
            jax.ShapeDtypeStruct((1, 1), jnp.float32),
        ],
    )(ms2d, counts)


def _l2norm(t):
    return t / jnp.maximum(jnp.linalg.norm(t, axis=-1, keepdims=True), 1e-12)


def kernel(x, codebook):
    b, n, d = x.shape
    m = b * n
    xn = _l2norm(x.reshape(m, d))
    cn = _l2norm(codebook)
    idx, maxsim = _tc_argmax(xn, cn)
    zq, counts = _sc_gather_hist(cn, idx)
    loss, perp = _tc_scalars(maxsim.reshape(128, m // 128), counts, m, d)
    return zq.reshape(b, n, d), loss.reshape(()), perp.reshape(())


# speedup vs baseline: 1.0809x; 1.0809x over previous
import functools

import jax
import jax.numpy as jnp
from jax import lax
from jax.experimental import pallas as pl
from jax.experimental.pallas import tpu as pltpu
from jax.experimental.pallas import tpu_sc as plsc


def _tc_argmax(xn, cn, mt=256):
    m, d = xn.shape
    k_total = cn.shape[0]
    n_mt = m // mt

    def body(x_ref, cn_ref, idx_ref, ms_ref):
        sim = lax.dot_general(x_ref[...], cn_ref[...], (((1,), (1,)), ((), ())),
                              preferred_element_type=jnp.float32)
        tmax = jnp.max(sim, axis=1)
        targ = jnp.argmax(sim, axis=1).astype(jnp.int32)
        idx_ref[...] = targ
        ms_ref[...] = tmax

    return pl.pallas_call(
        body,
        grid=(n_mt,),
        in_specs=[
            pl.BlockSpec((mt, d), lambda mi: (mi, 0)),
            pl.BlockSpec((k_total, d), lambda mi: (0, 0)),
        ],
        out_specs=[
            pl.BlockSpec((mt,), lambda mi: (mi,)),
            pl.BlockSpec((mt,), lambda mi: (mi,)),
        ],
        out_shape=[
            jax.ShapeDtypeStruct((m,), jnp.int32),
            jax.ShapeDtypeStruct((m,), jnp.float32),
        ],
        compiler_params=pltpu.CompilerParams(
            vmem_limit_bytes=100 * 1024 * 1024),
    )(xn, cn)


def _sc_gather_hist(cn, idx):
    k_total, d = cn.shape
    m = idx.shape[0]
    info = plsc.get_sparse_core_info()
    nc, ns = info.num_cores, info.num_subcores
    nw = nc * ns
    rows_w = m // nw
    ch = 128
    n_ch = rows_w // ch
    hr, hc = k_total // 128, 128

    mesh = plsc.VectorSubcoreMesh(core_axis_name="c", subcore_axis_name="s")

    @functools.partial(
        pl.kernel,
        out_type=[
            jax.ShapeDtypeStruct((m, d), jnp.float32),
            jax.ShapeDtypeStruct((nc, hr, hc), jnp.float32),
        ],
        mesh=mesh,
        compiler_params=pltpu.CompilerParams(needs_layout_passes=False),
        scratch_types=[
            pltpu.VMEM((rows_w,), jnp.int32),
            pltpu.VMEM((ch, d), jnp.float32),
            pltpu.VMEM((ch, d), jnp.float32),
            pltpu.VMEM((hr, hc), jnp.float32),
            pltpu.VMEM((hr,), jnp.int32),
            pltpu.VMEM_SHARED((hr, hc), jnp.float32),
            pltpu.SemaphoreType.DMA,
            pltpu.SemaphoreType.DMA,
        ],
    )
    def sc_body(cn_hbm, idx_hbm, zq_hbm, cnt_hbm,
                idx_v, buf0, buf1, hist_v, rowid_v, shared_hist, sem0, sem1):
        ci = lax.axis_index("c")
        si = lax.axis_index("s")
        wid = si * nc + ci
        base = wid * rows_w

        pltpu.sync_copy(idx_hbm.at[pl.ds(base, rows_w)], idx_v)
        bufs = (buf0, buf1)
        sems = (sem0, sem1)
        cps = [pltpu.async_copy(cn_hbm.at[idx_v.at[pl.ds(0, ch)]], buf0, sem0),
               pltpu.async_copy(cn_hbm.at[idx_v.at[pl.ds(ch, ch)]], buf1, sem1)]

        zeros16 = jnp.zeros((16,), jnp.float32)

        def zero_body(t, _):
            r = t // (hc // 16)
            c = (t % (hc // 16)) * 16
            hist_v[r, pl.ds(c, 16)] = zeros16
            return 0

        lax.fori_loop(0, hr * (hc // 16), zero_body, 0)
        for j in range(hr // 16):
            rowid_v[pl.ds(j * 16, 16)] = lax.iota(jnp.int32, 16) + j * 16

        @pl.when(si == 0)
        def _():
            pltpu.sync_copy(hist_v, shared_hist)

        lane = lax.iota(jnp.int32, 16)
        ones16 = jnp.ones((16,), jnp.float32)

        def hist_body(v, _):
            vec = idx_v[pl.ds(v * 16, 16)]
            row = lax.shift_right_logical(vec, 7)
            col = lax.bitwise_and(vec, 127)
            for j in range(16):
                plsc.addupdate_scatter(hist_v, [row, col], ones16,
                                       mask=lane == j)
            return 0

        lax.fori_loop(0, rows_w // 16, hist_body, 0)

        for c in range(n_ch):
            cps[c % 2].wait()
            pltpu.sync_copy(bufs[c % 2], zq_hbm.at[pl.ds(base + c * ch, ch)])
            if c + 2 < n_ch:
                cps[c % 2] = pltpu.async_copy(
                    cn_hbm.at[idx_v.at[pl.ds((c + 2) * ch, ch)]],
                    bufs[c % 2], sems[c % 2])

        plsc.subcore_barrier()
        pltpu.sync_copy(hist_v, shared_hist.at[rowid_v], add=True)
        plsc.subcore_barrier()

        @pl.when(si == 0)
        def _():
            pltpu.sync_copy(shared_hist, cnt_hbm.at[ci])

    return sc_body(cn, idx)


def _tc_scalars(ms2d, counts, m, d):
    def body(ms_ref, cnt_ref, loss_ref, perp_ref):
        mean_s = jnp.sum(ms_ref[...]) * (1.0 / m)
        loss = 1.25 * (2.0 - 2.0 * mean_s) * (1.0 / d)
        loss_ref[...] = jnp.broadcast_to(loss, (1, 1))
        cnt = cnt_ref[0] + cnt_ref[1]
        e = cnt * (1.0 / m)
        ent = -jnp.sum(e * jnp.log(e + 1e-10))
        perp_ref[...] = jnp.broadcast_to(jnp.exp(ent), (1, 1))

    return pl.pallas_call(
        body,
        out_shape=[
            jax.ShapeDtypeStruct((1, 1), jnp.float32),
            jax.ShapeDtypeStruct((1, 1), jnp.float32),
        ],
    )(ms2d, counts)


def _l2norm(t):
    return t / jnp.maximum(jnp.linalg.norm(t, axis=-1, keepdims=True), 1e-12)


def kernel(x, codebook):
    b, n, d = x.shape
    m = b * n
    xn = _l2norm(x.reshape(m, d))
    cn = _l2norm(codebook)
    idx, maxsim = _tc_argmax(xn.astype(jnp.bfloat16), cn.astype(jnp.bfloat16))
    zq, counts = _sc_gather_hist(cn, idx)
    loss, perp = _tc_scalars(maxsim.reshape(128, m // 128), counts, m, d)
    return zq.reshape(b, n, d), loss.reshape(()), perp.reshape(())


# speedup vs baseline: 1.0901x; 1.0086x over previous
import functools

import jax
import jax.numpy as jnp
from jax import lax
from jax.experimental import pallas as pl
from jax.experimental.pallas import tpu as pltpu
from jax.experimental.pallas import tpu_sc as plsc


def _tc_argmax(xn, cn, mt=512):
    m, d = xn.shape
    k_total = cn.shape[0]
    n_mt = m // mt

    def body(x_ref, cn_ref, idx_ref, ms_ref):
        sim = lax.dot_general(x_ref[...], cn_ref[...], (((1,), (1,)), ((), ())),
                              preferred_element_type=jnp.float32)
        tmax = jnp.max(sim, axis=1)
        targ = jnp.argmax(sim, axis=1).astype(jnp.int32)
        idx_ref[...] = targ
        ms_ref[...] = tmax

    return pl.pallas_call(
        body,
        grid=(n_mt,),
        in_specs=[
            pl.BlockSpec((mt, d), lambda mi: (mi, 0)),
            pl.BlockSpec((k_total, d), lambda mi: (0, 0)),
        ],
        out_specs=[
            pl.BlockSpec((mt,), lambda mi: (mi,)),
            pl.BlockSpec((mt,), lambda mi: (mi,)),
        ],
        out_shape=[
            jax.ShapeDtypeStruct((m,), jnp.int32),
            jax.ShapeDtypeStruct((m,), jnp.float32),
        ],
        compiler_params=pltpu.CompilerParams(
            vmem_limit_bytes=100 * 1024 * 1024),
    )(xn, cn)


def _sc_gather_hist(cn, idx):
    k_total, d = cn.shape
    m = idx.shape[0]
    info = plsc.get_sparse_core_info()
    nc, ns = info.num_cores, info.num_subcores
    nw = nc * ns
    rows_w = m // nw
    ch = 128
    n_ch = rows_w // ch
    hr, hc = k_total // 128, 128

    mesh = plsc.VectorSubcoreMesh(core_axis_name="c", subcore_axis_name="s")

    @functools.partial(
        pl.kernel,
        out_type=[
            jax.ShapeDtypeStruct((m, d), jnp.float32),
            jax.ShapeDtypeStruct((nc, hr, hc), jnp.float32),
        ],
        mesh=mesh,
        compiler_params=pltpu.CompilerParams(needs_layout_passes=False),
        scratch_types=[
            pltpu.VMEM((rows_w,), jnp.int32),
            pltpu.VMEM((ch, d), jnp.float32),
            pltpu.VMEM((ch, d), jnp.float32),
            pltpu.VMEM((hr, hc), jnp.float32),
            pltpu.VMEM((hr,), jnp.int32),
            pltpu.VMEM_SHARED((hr, hc), jnp.float32),
            pltpu.SemaphoreType.DMA,
            pltpu.SemaphoreType.DMA,
        ],
    )
    def sc_body(cn_hbm, idx_hbm, zq_hbm, cnt_hbm,
                idx_v, buf0, buf1, hist_v, rowid_v, shared_hist, sem0, sem1):
        ci = lax.axis_index("c")
        si = lax.axis_index("s")
        wid = si * nc + ci
        base = wid * rows_w

        pltpu.sync_copy(idx_hbm.at[pl.ds(base, rows_w)], idx_v)
        bufs = (buf0, buf1)
        sems = (sem0, sem1)
        cps = [pltpu.async_copy(cn_hbm.at[idx_v.at[pl.ds(0, ch)]], buf0, sem0),
               pltpu.async_copy(cn_hbm.at[idx_v.at[pl.ds(ch, ch)]], buf1, sem1)]

        zeros16 = jnp.zeros((16,), jnp.float32)

        def zero_body(t, _):
            r = t // (hc // 16)
            c = (t % (hc // 16)) * 16
            hist_v[r, pl.ds(c, 16)] = zeros16
            return 0

        lax.fori_loop(0, hr * (hc // 16), zero_body, 0)
        for j in range(hr // 16):
            rowid_v[pl.ds(j * 16, 16)] = lax.iota(jnp.int32, 16) + j * 16

        @pl.when(si == 0)
        def _():
            pltpu.sync_copy(hist_v, shared_hist)

        lane = lax.iota(jnp.int32, 16)
        ones16 = jnp.ones((16,), jnp.float32)

        def hist_body(v, _):
            vec = idx_v[pl.ds(v * 16, 16)]
            row = lax.shift_right_logical(vec, 7)
            col = lax.bitwise_and(vec, 127)
            for j in range(16):
                plsc.addupdate_scatter(hist_v, [row, col], ones16,
                                       mask=lane == j)
            return 0

        lax.fori_loop(0, rows_w // 16, hist_body, 0)

        for c in range(n_ch):
            cps[c % 2].wait()
            pltpu.sync_copy(bufs[c % 2], zq_hbm.at[pl.ds(base + c * ch, ch)])
            if c + 2 < n_ch:
                cps[c % 2] = pltpu.async_copy(
                    cn_hbm.at[idx_v.at[pl.ds((c + 2) * ch, ch)]],
                    bufs[c % 2], sems[c % 2])

        plsc.subcore_barrier()
        pltpu.sync_copy(hist_v, shared_hist.at[rowid_v], add=True)
        plsc.subcore_barrier()

        @pl.when(si == 0)
        def _():
            pltpu.sync_copy(shared_hist, cnt_hbm.at[ci])

    return sc_body(cn, idx)


def _tc_scalars(ms2d, counts, m, d):
    def body(ms_ref, cnt_ref, loss_ref, perp_ref):
        mean_s = jnp.sum(ms_ref[...]) * (1.0 / m)
        loss = 1.25 * (2.0 - 2.0 * mean_s) * (1.0 / d)
        loss_ref[...] = jnp.broadcast_to(loss, (1, 1))
        cnt = cnt_ref[0] + cnt_ref[1]
        e = cnt * (1.0 / m)
        ent = -jnp.sum(e * jnp.log(e + 1e-10))
        perp_ref[...] = jnp.broadcast_to(jnp.exp(ent), (1, 1))

    return pl.pallas_call(
        body,
        out_shape=[
            jax.ShapeDtypeStruct((1, 1), jnp.float32),
            jax.ShapeDtypeStruct((1, 1), jnp.float32),
        ],
    )(ms2d, counts)


def _l2norm(t):
    return t / jnp.maximum(jnp.linalg.norm(t, axis=-1, keepdims=True), 1e-12)


def kernel(x, codebook):
    b, n, d = x.shape
    m = b * n
    xn = _l2norm(x.reshape(m, d))
    cn = _l2norm(codebook)
    idx, maxsim = _tc_argmax(xn.astype(jnp.bfloat16), cn.astype(jnp.bfloat16))
    zq, counts = _sc_gather_hist(cn, idx)
    loss, perp = _tc_scalars(maxsim.reshape(128, m // 128), counts, m, d)
    return zq.reshape(b, n, d), loss.reshape(()), perp.reshape(())
